# baseline (speedup 1.0000x reference)
import jax
import jax.numpy as jnp
from jax.experimental import pallas as pl
from jax.experimental.pallas import tpu as pltpu

N_EDGES = 20000
D = 256
BLK = 2000
NB = N_EDGES // BLK


def _mp_kernel(x_ref, w3t_ref, w3b_ref, b3_ref, out_ref, pooled_scr, cvec_scr):
    p = pl.program_id(0)
    j = pl.program_id(1)

    @pl.when(p == 0)
    def _phase_max():
        blk_max = jnp.max(x_ref[...], axis=0, keepdims=True)

        @pl.when(j == 0)
        def _():
            pooled_scr[...] = blk_max

        @pl.when(j > 0)
        def _():
            pooled_scr[...] = jnp.maximum(pooled_scr[...], blk_max)

    @pl.when(p == 1)
    def _phase_gemm():
        @pl.when(j == 0)
        def _():
            cvec_scr[...] = (
                jnp.dot(pooled_scr[...], w3b_ref[...],
                        preferred_element_type=jnp.float32)
                + b3_ref[...]
            )

        y = jnp.dot(x_ref[...], w3t_ref[...],
                    preferred_element_type=jnp.float32) + cvec_scr[...]
        out_ref[...] = jnp.maximum(y, 0.0)


def kernel(edge_pred, edge_corner, all_corners, edge_x, image_x, W3, b3,
           interpret=False):
    del edge_pred, edge_corner, all_corners, image_x
    w3t = W3[:D, :]
    w3b = W3[D:, :]
    b3_2d = b3.reshape(1, D)

    out = pl.pallas_call(
        _mp_kernel,
        grid=(2, NB),
        in_specs=[
            pl.BlockSpec((BLK, D), lambda p, j: (j, 0)),
            pl.BlockSpec((D, D), lambda p, j: (0, 0)),
            pl.BlockSpec((D, D), lambda p, j: (0, 0)),
            pl.BlockSpec((1, D), lambda p, j: (0, 0)),
        ],
        out_specs=pl.BlockSpec((BLK, D), lambda p, j: (p * j, 0)),
        out_shape=jax.ShapeDtypeStruct((N_EDGES, D), jnp.float32),
        scratch_shapes=[
            pltpu.VMEM((1, D), jnp.float32),
            pltpu.VMEM((1, D), jnp.float32),
        ],
        interpret=interpret,
    )(edge_x, w3t, w3b, b3_2d)
    return out


# speedup vs baseline: 1.3658x; 1.3658x over previous
import jax
import jax.numpy as jnp
from jax.experimental import pallas as pl
from jax.experimental.pallas import tpu as pltpu

N_EDGES = 20000
D = 256
BLK = 2000
NB = N_EDGES // BLK


def _mp_kernel(x_ref, w3t_ref, w3b_ref, b3_ref, out_ref, cvec_scr):
    j = pl.program_id(0)

    @pl.when(j == 0)
    def _():
        pooled = jnp.max(x_ref[...], axis=0, keepdims=True)
        cvec_scr[...] = (
            jnp.dot(pooled, w3b_ref[...], preferred_element_type=jnp.float32)
            + b3_ref[...]
        )

    xblk = x_ref[pl.ds(j * BLK, BLK), :]
    y = jnp.dot(xblk, w3t_ref[...],
                preferred_element_type=jnp.float32) + cvec_scr[...]
    out_ref[...] = jnp.maximum(y, 0.0)


def kernel(edge_pred, edge_corner, all_corners, edge_x, image_x, W3, b3,
           interpret=False):
    del edge_pred, edge_corner, all_corners, image_x
    w3t = W3[:D, :]
    w3b = W3[D:, :]
    b3_2d = b3.reshape(1, D)

    out = pl.pallas_call(
        _mp_kernel,
        grid=(NB,),
        in_specs=[
            pl.BlockSpec((N_EDGES, D), lambda j: (0, 0)),
            pl.BlockSpec((D, D), lambda j: (0, 0)),
            pl.BlockSpec((D, D), lambda j: (0, 0)),
            pl.BlockSpec((1, D), lambda j: (0, 0)),
        ],
        out_specs=pl.BlockSpec((BLK, D), lambda j: (j, 0)),
        out_shape=jax.ShapeDtypeStruct((N_EDGES, D), jnp.float32),
        scratch_shapes=[
            pltpu.VMEM((1, D), jnp.float32),
        ],
        interpret=interpret,
    )(edge_x, w3t, w3b, b3_2d)
    return out
